# Initial kernel scaffold; baseline (speedup 1.0000x reference)
import jax
import jax.numpy as jnp
from jax.experimental import pallas as pl


def kernel(feats, proposals_xywh, W_cls, b_cls, W_reg, b_reg):
    raise NotImplementedError("write your pallas kernel here")


# speedup vs baseline: 1.0480x; 1.0480x over previous
import jax
import jax.numpy as jnp
from jax.experimental import pallas as pl

N = 5000
D = 4096
C = 81
TILE = 200


def _head_kernel(f_ref, p_ref, wc_ref, bc_ref, wr_ref, br_ref,
                 cls_ref, box_ref):
    f = f_ref[...]
    cls = jnp.dot(f, wc_ref[...], preferred_element_type=jnp.float32)
    cls_ref[...] = cls + bc_ref[...]

    deltas = jnp.dot(f, wr_ref[...], preferred_element_type=jnp.float32)
    deltas = deltas + br_ref[...]
    p = p_ref[...]
    px, py = p[:, 0:1], p[:, 1:2]
    pw, ph = p[:, 2:3], p[:, 3:4]
    x = deltas[:, 0:1] * pw + px
    y = deltas[:, 1:2] * ph + py
    ew = jnp.exp(deltas[:, 2:3])
    w = ew * pw
    h = ew * ph
    box_ref[...] = jnp.concatenate([x, y, w, h], axis=1)


def kernel(feats, proposals_xywh, W_cls, b_cls, W_reg, b_reg):
    wc_t = W_cls.T
    wr_t = W_reg.T
    bc = b_cls.reshape(1, C)
    br = b_reg.reshape(1, 4)
    grid = (N // TILE,)
    cls_out, box_out = pl.pallas_call(
        _head_kernel,
        grid=grid,
        in_specs=[
            pl.BlockSpec((TILE, D), lambda i: (i, 0)),
            pl.BlockSpec((TILE, 4), lambda i: (i, 0)),
            pl.BlockSpec((D, C), lambda i: (0, 0)),
            pl.BlockSpec((1, C), lambda i: (0, 0)),
            pl.BlockSpec((D, 4), lambda i: (0, 0)),
            pl.BlockSpec((1, 4), lambda i: (0, 0)),
        ],
        out_specs=[
            pl.BlockSpec((TILE, C), lambda i: (i, 0)),
            pl.BlockSpec((TILE, 4), lambda i: (i, 0)),
        ],
        out_shape=[
            jax.ShapeDtypeStruct((N, C), jnp.float32),
            jax.ShapeDtypeStruct((N, 4), jnp.float32),
        ],
    )(feats, proposals_xywh, wc_t, bc, wr_t, br)
    return (cls_out, box_out)


# speedup vs baseline: 1.1949x; 1.1402x over previous
import jax
import jax.numpy as jnp
from jax.experimental import pallas as pl
from jax.experimental.pallas import tpu as pltpu

N = 5000
D = 4096
C = 81
CW = C + 4
TILE = 200


def _head_kernel(f_ref, p_ref, w_ref, b_ref, cls_ref, box_ref):
    f = f_ref[...]
    acc = jnp.dot(f, w_ref[...], preferred_element_type=jnp.float32)
    acc = acc + b_ref[...]
    cls_ref[...] = acc[:, :C]

    deltas = acc[:, C:CW]
    p = p_ref[...]
    px, py = p[:, 0:1], p[:, 1:2]
    pw, ph = p[:, 2:3], p[:, 3:4]
    x = deltas[:, 0:1] * pw + px
    y = deltas[:, 1:2] * ph + py
    ew = jnp.exp(deltas[:, 2:3])
    w = ew * pw
    h = ew * ph
    box_ref[...] = jnp.concatenate([x, y, w, h], axis=1)


def kernel(feats, proposals_xywh, W_cls, b_cls, W_reg, b_reg):
    w_t = jnp.concatenate([W_cls, W_reg], axis=0).T
    b = jnp.concatenate([b_cls, b_reg]).reshape(1, CW)
    grid = (N // TILE,)
    cls_out, box_out = pl.pallas_call(
        _head_kernel,
        grid=grid,
        in_specs=[
            pl.BlockSpec((TILE, D), lambda i: (i, 0)),
            pl.BlockSpec((TILE, 4), lambda i: (i, 0)),
            pl.BlockSpec((D, CW), lambda i: (0, 0)),
            pl.BlockSpec((1, CW), lambda i: (0, 0)),
        ],
        out_specs=[
            pl.BlockSpec((TILE, C), lambda i: (i, 0)),
            pl.BlockSpec((TILE, 4), lambda i: (i, 0)),
        ],
        out_shape=[
            jax.ShapeDtypeStruct((N, C), jnp.float32),
            jax.ShapeDtypeStruct((N, 4), jnp.float32),
        ],
        compiler_params=pltpu.CompilerParams(
            dimension_semantics=("parallel",)),
    )(feats, proposals_xywh, w_t, b)
    return (cls_out, box_out)


# speedup vs baseline: 1.3552x; 1.1342x over previous
import jax
import jax.numpy as jnp
from jax.experimental import pallas as pl
from jax.experimental.pallas import tpu as pltpu

N = 5000
D = 4096
C = 81
CW = C + 4
TILE = 1000


def _head_kernel(f_ref, p_ref, w_ref, b_ref, cls_ref, box_ref):
    f = f_ref[...]
    acc = jnp.dot(f, w_ref[...], preferred_element_type=jnp.float32)
    acc = acc + b_ref[...]
    cls_ref[...] = acc[:, :C]

    deltas = acc[:, C:CW]
    p = p_ref[...]
    px, py = p[:, 0:1], p[:, 1:2]
    pw, ph = p[:, 2:3], p[:, 3:4]
    x = deltas[:, 0:1] * pw + px
    y = deltas[:, 1:2] * ph + py
    ew = jnp.exp(deltas[:, 2:3])
    w = ew * pw
    h = ew * ph
    box_ref[...] = jnp.concatenate([x, y, w, h], axis=1)


def kernel(feats, proposals_xywh, W_cls, b_cls, W_reg, b_reg):
    w_t = jnp.concatenate([W_cls, W_reg], axis=0).T
    b = jnp.concatenate([b_cls, b_reg]).reshape(1, CW)
    grid = (N // TILE,)
    cls_out, box_out = pl.pallas_call(
        _head_kernel,
        grid=grid,
        in_specs=[
            pl.BlockSpec((TILE, D), lambda i: (i, 0)),
            pl.BlockSpec((TILE, 4), lambda i: (i, 0)),
            pl.BlockSpec((D, CW), lambda i: (0, 0)),
            pl.BlockSpec((1, CW), lambda i: (0, 0)),
        ],
        out_specs=[
            pl.BlockSpec((TILE, C), lambda i: (i, 0)),
            pl.BlockSpec((TILE, 4), lambda i: (i, 0)),
        ],
        out_shape=[
            jax.ShapeDtypeStruct((N, C), jnp.float32),
            jax.ShapeDtypeStruct((N, 4), jnp.float32),
        ],
        compiler_params=pltpu.CompilerParams(
            dimension_semantics=("parallel",)),
    )(feats, proposals_xywh, w_t, b)
    return (cls_out, box_out)
